# baseline (speedup 1.0000x reference)
import functools

import jax
import jax.numpy as jnp
from jax import lax
from jax.experimental import pallas as pl
from jax.experimental.pallas import tpu as pltpu
from jax.experimental.pallas import tpu_sc as plsc

B = 16384
L = 20
EMB = 32
TV = 100001
NC = 2
NS = 16
NW = NC * NS
BPW = B // NW
C = 64
NCHUNK = BPW // C
QH = B // 4


def _splat(vec, lane):
    lanes = jnp.full((16,), lane, dtype=jnp.int32)
    dnums = lax.GatherDimensionNumbers(
        offset_dims=(), collapsed_slice_dims=(0,), start_index_map=(0,))
    return lax.gather(vec, lanes[:, None], dnums, slice_sizes=(1,),
                      mode=lax.GatherScatterMode.PROMISE_IN_BOUNDS)


def _tree_sum(vals):
    vals = list(vals)
    while len(vals) > 1:
        nxt = [a + b for a, b in zip(vals[::2], vals[1::2])]
        if len(vals) % 2:
            nxt.append(vals[-1])
        vals = nxt
    return vals[0]


def _make_title_kernel():
    mesh = plsc.VectorSubcoreMesh(core_axis_name="c", subcore_axis_name="s")

    @functools.partial(
        pl.kernel,
        mesh=mesh,
        out_type=jax.ShapeDtypeStruct((EMB, B), jnp.float32),
        scratch_types=[
            pltpu.VMEM((TV,), jnp.float32),
            pltpu.VMEM((2, QH), jnp.int32),
            pltpu.VMEM((2, QH), jnp.float32),
            pltpu.SemaphoreType.DMA,
            pltpu.SemaphoreType.DMA,
        ],
        compiler_params=pltpu.CompilerParams(use_tc_tiling_on_sc=False,
                                             needs_layout_passes=False),
    )
    def kern(tab_t_h, tidx_h, out_h, row_v, idx_v, ob_v, sem, osem):
        wid = lax.axis_index("s") * NC + lax.axis_index("c")
        cp_row = pltpu.async_copy(tab_t_h.at[wid], row_v, sem)
        inflight = pltpu.async_copy(tidx_h.at[pl.ds(0, QH)], idx_v.at[0], sem)
        cp_row.wait()
        out_cps = [None, None]
        for q in range(4):
            buf = q % 2
            nxt = inflight
            if q + 1 < 4:
                nxt = pltpu.async_copy(
                    tidx_h.at[pl.ds((q + 1) * QH, QH)],
                    idx_v.at[1 - buf], sem)
            inflight.wait()
            inflight = nxt
            if out_cps[buf] is not None:
                out_cps[buf].wait()

            @plsc.parallel_loop(0, QH // 16)
            def g_body(g):
                iv = idx_v[buf, pl.ds(g * 16, 16)]
                ob_v[buf, pl.ds(g * 16, 16)] = plsc.load_gather(row_v, [iv])

            out_cps[buf] = pltpu.async_copy(
                ob_v.at[buf], out_h.at[wid, pl.ds(q * QH, QH)], osem)
        for cp in out_cps:
            if cp is not None:
                cp.wait()

    return kern


def _make_text_kernel():
    mesh = plsc.VectorSubcoreMesh(core_axis_name="c", subcore_axis_name="s")

    @functools.partial(
        pl.kernel,
        mesh=mesh,
        out_type=jax.ShapeDtypeStruct((2 * EMB, B), jnp.float32),
        scratch_types=[
            pltpu.VMEM((L * NCHUNK, C), jnp.int32),
            pltpu.VMEM((2, L, C, EMB), jnp.float32),
            pltpu.VMEM((2, EMB, C + 1), jnp.float32),
            pltpu.VMEM((BPW,), jnp.float32),
            pltpu.VMEM((BPW,), jnp.float32),
            pltpu.VMEM((1, EMB), jnp.float32),
            [pltpu.SemaphoreType.DMA] * 2,
            pltpu.SemaphoreType.DMA,
            pltpu.SemaphoreType.DMA,
        ],
        compiler_params=pltpu.CompilerParams(use_tc_tiling_on_sc=False,
                                             needs_layout_passes=False),
    )
    def kern(tok3_h, text_tab_h, title_t_h, out_h,
             ids_v, rows_v, outv, inv_v, s2_v, t0_v,
             gsems, ssem, osem):
        wid = lax.axis_index("s") * NC + lax.axis_index("c")
        base_w = wid * BPW

        tmerge = pltpu.async_copy(
            title_t_h.at[:, pl.ds(base_w, BPW)],
            out_h.at[pl.ds(0, EMB), pl.ds(base_w, BPW)], osem)

        stage = [pltpu.async_copy(
            tok3_h.at[l, pl.ds(wid * NCHUNK, NCHUNK)],
            ids_v.at[pl.ds(l * NCHUNK, NCHUNK)], ssem) for l in range(L)]
        stage.append(pltpu.async_copy(text_tab_h.at[pl.ds(0, 1)], t0_v, ssem))
        for cp in stage:
            cp.wait()

        t0a = t0_v[0, pl.ds(0, 16)]
        t0b = t0_v[0, pl.ds(16, 16)]

        def fire(ci):
            buf = ci % 2
            return [pltpu.async_copy(
                text_tab_h.at[ids_v.at[l * NCHUNK + ci]],
                rows_v.at[buf, l], gsems[buf]) for l in range(L)]

        inflight = fire(0)

        def count_body(g, carry):
            ci = g >> 2
            off = (g & 3) * 16
            n = jnp.zeros((16,), jnp.float32)
            for l in range(L):
                idv = ids_v[l * NCHUNK + ci, pl.ds(off, 16)]
                n = n + jnp.where(idv != 0, jnp.float32(1), jnp.float32(0))
            inv = jnp.float32(1) / jnp.maximum(n, jnp.float32(1))
            base = ci * C + off
            inv_v[pl.ds(base, 16)] = inv
            s2_v[pl.ds(base, 16)] = (jnp.float32(L) - n) * inv
            return carry
        lax.fori_loop(0, BPW // 16, count_body, 0)

        iota16 = lax.iota(jnp.int32, 16)

        out_cps = [None, None]
        for ci in range(NCHUNK):
            buf = ci % 2
            nxt = inflight if ci + 1 == NCHUNK else fire(ci + 1)
            for cp in inflight:
                cp.wait()
            inflight = nxt

            if out_cps[buf] is not None:
                out_cps[buf].wait()

            @plsc.parallel_loop(0, C)
            def row_body(b):
                r = ci * C + b
                lane = r & 15
                goff = r - lane
                s1 = _splat(inv_v[pl.ds(goff, 16)], lane)
                s2 = _splat(s2_v[pl.ds(goff, 16)], lane)
                bvec = jnp.full((16,), b, dtype=jnp.int32)
                for j in range(2):
                    js = pl.ds(j * 16, 16)
                    rows16 = iota16 + (j * 16)
                    acc = _tree_sum(
                        rows_v[buf, l, b, js] for l in range(L))
                    t0j = t0a if j == 0 else t0b
                    plsc.store_scatter(
                        outv.at[buf], [rows16, bvec], acc * s1 - s2 * t0j)

            out_cps[buf] = pltpu.async_copy(
                outv.at[buf, :, pl.ds(0, C)],
                out_h.at[pl.ds(EMB, EMB), pl.ds(base_w + ci * C, C)], osem)

        for cp in out_cps:
            if cp is not None:
                cp.wait()
        tmerge.wait()

    return kern


_tkern = _make_title_kernel()
_kern = _make_text_kernel()


@jax.jit
def kernel(title_idx, token_ids, title_table, text_table):
    tok3 = token_ids.T.reshape(L, B // C, C)
    title_t = _tkern(title_table.T, title_idx)
    out_t = _kern(tok3, text_table, title_t)
    return out_t.T


# speedup vs baseline: 1.7764x; 1.7764x over previous
import functools

import jax
import jax.numpy as jnp
from jax import lax
from jax.experimental import pallas as pl
from jax.experimental.pallas import tpu as pltpu
from jax.experimental.pallas import tpu_sc as plsc

B = 16384
L = 20
EMB = 32
TV = 100001
NC = 2
NS = 16
NW = NC * NS
BPW = B // NW
C = 64
NCHUNK = BPW // C
QH = B // 4


def _splat(vec, lane):
    lanes = jnp.full((16,), lane, dtype=jnp.int32)
    dnums = lax.GatherDimensionNumbers(
        offset_dims=(), collapsed_slice_dims=(0,), start_index_map=(0,))
    return lax.gather(vec, lanes[:, None], dnums, slice_sizes=(1,),
                      mode=lax.GatherScatterMode.PROMISE_IN_BOUNDS)


def _tree_sum(vals):
    vals = list(vals)
    while len(vals) > 1:
        nxt = [a + b for a, b in zip(vals[::2], vals[1::2])]
        if len(vals) % 2:
            nxt.append(vals[-1])
        vals = nxt
    return vals[0]


def _make_title_kernel():
    mesh = plsc.VectorSubcoreMesh(core_axis_name="c", subcore_axis_name="s")

    @functools.partial(
        pl.kernel,
        mesh=mesh,
        out_type=jax.ShapeDtypeStruct((EMB, B), jnp.float32),
        scratch_types=[
            pltpu.VMEM((TV,), jnp.float32),
            pltpu.VMEM((2, QH), jnp.int32),
            pltpu.VMEM((2, QH), jnp.float32),
            pltpu.SemaphoreType.DMA,
            pltpu.SemaphoreType.DMA,
        ],
        compiler_params=pltpu.CompilerParams(use_tc_tiling_on_sc=True,
                                             needs_layout_passes=False),
    )
    def kern(tab_t_h, tidx_h, out_h, row_v, idx_v, ob_v, sem, osem):
        wid = lax.axis_index("s") * NC + lax.axis_index("c")
        cp_row = pltpu.async_copy(tab_t_h.at[wid], row_v, sem)
        inflight = pltpu.async_copy(tidx_h.at[pl.ds(0, QH)], idx_v.at[0], sem)
        cp_row.wait()
        out_cps = [None, None]
        for q in range(4):
            buf = q % 2
            nxt = inflight
            if q + 1 < 4:
                nxt = pltpu.async_copy(
                    tidx_h.at[pl.ds((q + 1) * QH, QH)],
                    idx_v.at[1 - buf], sem)
            inflight.wait()
            inflight = nxt
            if out_cps[buf] is not None:
                out_cps[buf].wait()

            @plsc.parallel_loop(0, QH // 16)
            def g_body(g):
                iv = idx_v[buf, pl.ds(g * 16, 16)]
                ob_v[buf, pl.ds(g * 16, 16)] = plsc.load_gather(row_v, [iv])

            out_cps[buf] = pltpu.async_copy(
                ob_v.at[buf], out_h.at[wid, pl.ds(q * QH, QH)], osem)
        for cp in out_cps:
            if cp is not None:
                cp.wait()

    return kern


def _make_text_kernel():
    mesh = plsc.VectorSubcoreMesh(core_axis_name="c", subcore_axis_name="s")

    @functools.partial(
        pl.kernel,
        mesh=mesh,
        out_type=jax.ShapeDtypeStruct((EMB, B), jnp.float32),
        scratch_types=[
            pltpu.VMEM((L * NCHUNK, C), jnp.int32),
            pltpu.VMEM((2, L, C, EMB), jnp.float32),
            pltpu.VMEM((2, EMB, C + 1), jnp.float32),
            pltpu.VMEM((BPW,), jnp.float32),
            pltpu.VMEM((BPW,), jnp.float32),
            pltpu.VMEM((1, EMB), jnp.float32),
            [pltpu.SemaphoreType.DMA] * 2,
            pltpu.SemaphoreType.DMA,
            pltpu.SemaphoreType.DMA,
        ],
        compiler_params=pltpu.CompilerParams(use_tc_tiling_on_sc=False,
                                             needs_layout_passes=False),
    )
    def kern(tok3_h, text_tab_h, out_h,
             ids_v, rows_v, outv, inv_v, s2_v, t0_v,
             gsems, ssem, osem):
        wid = lax.axis_index("s") * NC + lax.axis_index("c")
        base_w = wid * BPW

        stage = [pltpu.async_copy(
            tok3_h.at[l, pl.ds(wid * NCHUNK, NCHUNK)],
            ids_v.at[pl.ds(l * NCHUNK, NCHUNK)], ssem) for l in range(L)]
        stage.append(pltpu.async_copy(text_tab_h.at[pl.ds(0, 1)], t0_v, ssem))
        for cp in stage:
            cp.wait()

        t0a = t0_v[0, pl.ds(0, 16)]
        t0b = t0_v[0, pl.ds(16, 16)]

        def fire(ci):
            buf = ci % 2
            return [pltpu.async_copy(
                text_tab_h.at[ids_v.at[l * NCHUNK + ci]],
                rows_v.at[buf, l], gsems[buf]) for l in range(L)]

        inflight = fire(0)

        def count_body(g, carry):
            ci = g >> 2
            off = (g & 3) * 16
            n = jnp.zeros((16,), jnp.float32)
            for l in range(L):
                idv = ids_v[l * NCHUNK + ci, pl.ds(off, 16)]
                n = n + jnp.where(idv != 0, jnp.float32(1), jnp.float32(0))
            inv = jnp.float32(1) / jnp.maximum(n, jnp.float32(1))
            base = ci * C + off
            inv_v[pl.ds(base, 16)] = inv
            s2_v[pl.ds(base, 16)] = (jnp.float32(L) - n) * inv
            return carry
        lax.fori_loop(0, BPW // 16, count_body, 0)

        iota16 = lax.iota(jnp.int32, 16)

        out_cps = [None, None]
        for ci in range(NCHUNK):
            buf = ci % 2
            nxt = inflight if ci + 1 == NCHUNK else fire(ci + 1)
            for cp in inflight:
                cp.wait()
            inflight = nxt

            if out_cps[buf] is not None:
                out_cps[buf].wait()

            @plsc.parallel_loop(0, C)
            def row_body(b):
                r = ci * C + b
                lane = r & 15
                goff = r - lane
                s1 = _splat(inv_v[pl.ds(goff, 16)], lane)
                s2 = _splat(s2_v[pl.ds(goff, 16)], lane)
                bvec = jnp.full((16,), b, dtype=jnp.int32)
                for j in range(2):
                    js = pl.ds(j * 16, 16)
                    rows16 = iota16 + (j * 16)
                    acc = _tree_sum(
                        rows_v[buf, l, b, js] for l in range(L))
                    t0j = t0a if j == 0 else t0b
                    plsc.store_scatter(
                        outv.at[buf], [rows16, bvec], acc * s1 - s2 * t0j)

            out_cps[buf] = pltpu.async_copy(
                outv.at[buf, :, pl.ds(0, C)],
                out_h.at[:, pl.ds(base_w + ci * C, C)], osem)

        for cp in out_cps:
            if cp is not None:
                cp.wait()

    return kern


_tkern = _make_title_kernel()
_kern = _make_text_kernel()


@jax.jit
def kernel(title_idx, token_ids, title_table, text_table):
    tok3 = token_ids.T.reshape(L, B // C, C)
    title_t = _tkern(title_table.T, title_idx)
    text_t = _kern(tok3, text_table)
    return jnp.concatenate([title_t, text_t], axis=0).T
